# baseline (speedup 1.0000x reference)
import functools

import jax
import jax.numpy as jnp
from jax import lax
from jax.experimental import pallas as pl
from jax.experimental.pallas import tpu as pltpu
from jax.experimental.pallas import tpu_sc as plsc

N_NODES = 10000
N_PAD = 10240
N_EDGES = 320000
FEAT = 128

NC, NS, L = 2, 16, 16
NW = NC * NS

_mesh = functools.partial(
    plsc.VectorSubcoreMesh, core_axis_name="c", subcore_axis_name="s")


_EPT_A = N_EDGES // NS
_HALF = N_PAD // NC


@functools.partial(
    pl.kernel,
    out_type=jax.ShapeDtypeStruct((N_PAD,), jnp.int32),
    mesh=_mesh(),
    compiler_params=pltpu.CompilerParams(needs_layout_passes=False),
    scratch_types=[
        pltpu.VMEM((_EPT_A,), jnp.int32),
        pltpu.VMEM((N_PAD,), jnp.int32),
        pltpu.VMEM((_HALF // NS,), jnp.int32),
        pltpu.VMEM_SHARED((NS * N_PAD,), jnp.int32),
    ],
)
def _deg_kernel(dst_hbm, cnt_hbm, dst_v, acc_v, red_v, shared):
    cid = lax.axis_index("c")
    sid = lax.axis_index("s")
    zeros = jnp.zeros((L,), jnp.int32)

    def zero_body(i, _):
        acc_v[pl.ds(i * L, L)] = zeros

    lax.fori_loop(0, N_PAD // L, zero_body, None, unroll=8)

    pltpu.sync_copy(dst_hbm.at[pl.ds(sid * _EPT_A, _EPT_A)], dst_v)
    ones = jnp.ones((L,), jnp.int32)

    def body(i, _):
        idx = dst_v[pl.ds(i * L, L)]
        plsc.addupdate_scatter(acc_v, [idx], ones)

    lax.fori_loop(0, _EPT_A // L, body, None, unroll=4)

    pltpu.sync_copy(acc_v, shared.at[pl.ds(sid * N_PAD, N_PAD)])
    plsc.subcore_barrier()

    chunk = _HALF // NS
    base = cid * _HALF + sid * chunk

    def zero_red(i, _):
        red_v[pl.ds(i * L, L)] = zeros

    lax.fori_loop(0, chunk // L, zero_red, None, unroll=8)

    def red_body(t, _):
        pltpu.sync_copy(shared.at[pl.ds(t * N_PAD + base, chunk)],
                        dst_v.at[pl.ds(0, chunk)])

        def add_body(i, _):
            red_v[pl.ds(i * L, L)] = red_v[pl.ds(i * L, L)] + dst_v[pl.ds(i * L, L)]

        lax.fori_loop(0, chunk // L, add_body, None, unroll=8)

    lax.fori_loop(0, NS, red_body, None)
    pltpu.sync_copy(red_v, cnt_hbm.at[pl.ds(base, chunk)])


_EP_PAD = 327680
_EPC = _EP_PAD // NC
_EPT_B = _EPC // NS
_BB = 128
_NB = _EPT_B // _BB
_ROWS_PER_TILE = N_PAD // NS


@functools.partial(
    pl.kernel,
    out_type=jax.ShapeDtypeStruct((NC, N_PAD, FEAT), jnp.float32),
    mesh=_mesh(),
    compiler_params=pltpu.CompilerParams(needs_layout_passes=False),
    scratch_types=[
        pltpu.VMEM((_EPT_B + _BB,), jnp.int32),
        pltpu.VMEM((_BB,), jnp.int32),
        pltpu.VMEM((_BB,), jnp.int32),
        pltpu.VMEM((_BB,), jnp.int32),
        pltpu.VMEM((_BB,), jnp.int32),
        pltpu.VMEM((_BB, FEAT), jnp.float32),
        pltpu.VMEM((_BB, FEAT), jnp.float32),
        pltpu.VMEM_SHARED((N_PAD, FEAT), jnp.float32),
        pltpu.SemaphoreType.DMA,
        pltpu.SemaphoreType.DMA,
        pltpu.SemaphoreType.DMA,
    ],
)
def _msg_kernel(pk_hbm, p_hbm, out_hbm, pk_v, srcb0, srcb1, dstb0, dstb1,
                rows0, rows1, acc, sem_i, sem_g0, sem_g1):
    cid = lax.axis_index("c")
    sid = lax.axis_index("s")
    ebase = cid * _EPC + sid * _EPT_B

    h_pk = pltpu.async_copy(pk_hbm.at[pl.ds(ebase, _EPT_B)],
                            pk_v.at[pl.ds(0, _EPT_B)], sem_i)

    izeros = jnp.zeros((L,), jnp.int32)
    for j in range(_BB // L):
        pk_v[pl.ds(_EPT_B + j * L, L)] = izeros

    zeros = jnp.zeros((L,), jnp.float32)

    def zero_body(i, _):
        rows0[i // (FEAT // L), pl.ds((i % (FEAT // L)) * L, L)] = zeros

    lax.fori_loop(0, _BB * FEAT // L, zero_body, None, unroll=8)

    row0 = sid * _ROWS_PER_TILE
    for j in range(_ROWS_PER_TILE // _BB):
        pltpu.sync_copy(rows0, acc.at[pl.ds(row0 + j * _BB, _BB)])
    h_pk.wait()
    plsc.subcore_barrier()

    def unpack(b, sbuf, dbuf):
        def u(m, _):
            v = pk_v[pl.ds(b * _BB + m * L, L)]
            sbuf[pl.ds(m * L, L)] = v & 0xFFFF
            dbuf[pl.ds(m * L, L)] = v >> 16

        lax.fori_loop(0, _BB // L, u, None, unroll=8)

    unpack(0, srcb0, dstb0)
    pltpu.async_copy(p_hbm.at[srcb0], rows0, sem_g0).wait()

    def body(jo, _):
        b0 = jo * 2
        unpack(b0 + 1, srcb1, dstb1)
        h = pltpu.async_copy(p_hbm.at[srcb1], rows1, sem_g1)
        pltpu.sync_copy(rows0, acc.at[dstb0], add=True)
        h.wait()
        unpack(b0 + 2, srcb0, dstb0)
        h = pltpu.async_copy(p_hbm.at[srcb0], rows0, sem_g0)
        pltpu.sync_copy(rows1, acc.at[dstb1], add=True)
        h.wait()

    lax.fori_loop(0, _NB // 2, body, None)
    plsc.subcore_barrier()
    pltpu.sync_copy(acc.at[pl.ds(row0, _ROWS_PER_TILE)],
                    out_hbm.at[cid, pl.ds(row0, _ROWS_PER_TILE)])


_EPT_C = _EP_PAD // NS


@functools.partial(
    pl.kernel,
    out_type=jax.ShapeDtypeStruct((N_PAD,), jnp.float32),
    mesh=_mesh(),
    compiler_params=pltpu.CompilerParams(needs_layout_passes=False),
    scratch_types=[
        pltpu.VMEM((_EPT_C,), jnp.int32),
        pltpu.VMEM((N_PAD,), jnp.float32),
        pltpu.VMEM((N_PAD,), jnp.float32),
        pltpu.VMEM((_HALF // NS,), jnp.float32),
        pltpu.VMEM((_HALF // NS,), jnp.float32),
        pltpu.VMEM_SHARED((NS * N_PAD,), jnp.float32),
        pltpu.SemaphoreType.DMA,
    ],
)
def _scalar_kernel(pk_hbm, q_hbm, dinv_hbm, r_hbm, out_hbm,
                   pk_v, q_v, acc_v, red_v, aux_v, shared, sem_i):
    cid = lax.axis_index("c")
    sid = lax.axis_index("s")
    zeros = jnp.zeros((L,), jnp.float32)

    h_pk = pltpu.async_copy(pk_hbm.at[pl.ds(sid * _EPT_C, _EPT_C)], pk_v, sem_i)
    h_q = pltpu.async_copy(q_hbm, q_v, sem_i)

    def zero_body(i, _):
        acc_v[pl.ds(i * L, L)] = zeros

    lax.fori_loop(0, N_PAD // L, zero_body, None, unroll=8)
    h_pk.wait()
    h_q.wait()

    def body(i, _):
        v = pk_v[pl.ds(i * L, L)]
        s_idx = v & 0xFFFF
        d_idx = v >> 16
        vals = plsc.load_gather(q_v, [s_idx])
        plsc.addupdate_scatter(acc_v, [d_idx], vals)

    lax.fori_loop(0, _EPT_C // L, body, None, unroll=4)

    pltpu.sync_copy(acc_v, shared.at[pl.ds(sid * N_PAD, N_PAD)])
    plsc.subcore_barrier()

    chunk = _HALF // NS
    base = cid * _HALF + sid * chunk

    def zero_red(i, _):
        red_v[pl.ds(i * L, L)] = zeros

    lax.fori_loop(0, chunk // L, zero_red, None, unroll=8)

    def red_body(t, _):
        pltpu.sync_copy(shared.at[pl.ds(t * N_PAD + base, chunk)],
                        acc_v.at[pl.ds(0, chunk)])

        def add_body(i, _):
            red_v[pl.ds(i * L, L)] = red_v[pl.ds(i * L, L)] + acc_v[pl.ds(i * L, L)]

        lax.fori_loop(0, chunk // L, add_body, None, unroll=8)

    lax.fori_loop(0, NS, red_body, None)

    pltpu.sync_copy(dinv_hbm.at[pl.ds(base, chunk)], aux_v)

    def mul_body(i, _):
        red_v[pl.ds(i * L, L)] = red_v[pl.ds(i * L, L)] * aux_v[pl.ds(i * L, L)]

    lax.fori_loop(0, chunk // L, mul_body, None, unroll=8)
    pltpu.sync_copy(r_hbm.at[pl.ds(base, chunk)], aux_v)

    def add_r_body(i, _):
        red_v[pl.ds(i * L, L)] = red_v[pl.ds(i * L, L)] + aux_v[pl.ds(i * L, L)]

    lax.fori_loop(0, chunk // L, add_r_body, None, unroll=8)
    pltpu.sync_copy(red_v, out_hbm.at[pl.ds(base, chunk)])


_RB = 2048
_NRB = N_PAD // _RB


def _tc1_body(cnt_ref, x_ref, w1_ref, dinv_ref, p_ref):
    deg = cnt_ref[...].astype(jnp.float32) + 1.0
    dinv = lax.rsqrt(deg)
    dinv_ref[...] = dinv
    h = jnp.dot(x_ref[...], w1_ref[...], preferred_element_type=jnp.float32)
    p_ref[...] = dinv * h


def _tc1(cnt, x_pad, W1):
    return pl.pallas_call(
        _tc1_body,
        grid=(_NRB,),
        in_specs=[
            pl.BlockSpec((_RB, 1), lambda i: (i, 0)),
            pl.BlockSpec((_RB, FEAT), lambda i: (i, 0)),
            pl.BlockSpec((FEAT, FEAT), lambda i: (0, 0)),
        ],
        out_specs=[
            pl.BlockSpec((_RB, 1), lambda i: (i, 0)),
            pl.BlockSpec((_RB, FEAT), lambda i: (i, 0)),
        ],
        out_shape=[
            jax.ShapeDtypeStruct((N_PAD, 1), jnp.float32),
            jax.ShapeDtypeStruct((N_PAD, FEAT), jnp.float32),
        ],
    )(cnt, x_pad, W1)


def _tc2_body(s_ref, p_ref, dinv_ref, b1_ref, w2_ref, b2_ref, q_ref, r_ref):
    dinv = dinv_ref[...]
    s = s_ref[0] + s_ref[1] + p_ref[...]
    t = jnp.maximum(dinv * s + b1_ref[...], 0.0)
    z = jnp.dot(t, w2_ref[...], preferred_element_type=jnp.float32)
    q = dinv * z
    q_ref[...] = q
    r_ref[...] = dinv * q + b2_ref[0, 0]


def _tc2(s_parts, p, dinv, b1, W2, b2):
    return pl.pallas_call(
        _tc2_body,
        grid=(_NRB,),
        in_specs=[
            pl.BlockSpec((NC, _RB, FEAT), lambda i: (0, i, 0)),
            pl.BlockSpec((_RB, FEAT), lambda i: (i, 0)),
            pl.BlockSpec((_RB, 1), lambda i: (i, 0)),
            pl.BlockSpec((1, FEAT), lambda i: (0, 0)),
            pl.BlockSpec((FEAT, 1), lambda i: (0, 0)),
            pl.BlockSpec((1, 1), lambda i: (0, 0)),
        ],
        out_specs=[
            pl.BlockSpec((_RB, 1), lambda i: (i, 0)),
            pl.BlockSpec((_RB, 1), lambda i: (i, 0)),
        ],
        out_shape=[
            jax.ShapeDtypeStruct((N_PAD, 1), jnp.float32),
            jax.ShapeDtypeStruct((N_PAD, 1), jnp.float32),
        ],
    )(s_parts, p, dinv, b1, W2, b2)


def kernel(x, edge_index, W1, b1, W2, b2):
    edge_index = edge_index.astype(jnp.int32)
    src_idx = edge_index[0]
    dst_idx = edge_index[1]
    x_pad = jnp.pad(x, ((0, N_PAD - N_NODES), (0, 0)))

    src_pad = jnp.pad(src_idx, (0, _EP_PAD - N_EDGES))
    dst_pad = jnp.pad(dst_idx, (0, _EP_PAD - N_EDGES),
                      constant_values=N_PAD - 1)
    pk = jnp.bitwise_or(src_pad, jnp.left_shift(dst_pad, 16))

    cnt = _deg_kernel(dst_idx)
    dinv, p = _tc1(cnt.reshape(N_PAD, 1), x_pad, W1)
    s_parts = _msg_kernel(pk, p)
    q, r = _tc2(s_parts, p, dinv, b1.reshape(1, FEAT), W2, b2.reshape(1, 1))
    out = _scalar_kernel(pk, q.reshape(-1), dinv.reshape(-1), r.reshape(-1))
    return out[:N_NODES]


# speedup vs baseline: 1.1612x; 1.1612x over previous
import functools

import jax
import jax.numpy as jnp
from jax import lax
from jax.experimental import pallas as pl
from jax.experimental.pallas import tpu as pltpu
from jax.experimental.pallas import tpu_sc as plsc

N_NODES = 10000
N_PAD = 10240
N_EDGES = 320000
FEAT = 128

NC, NS, L = 2, 16, 16
NW = NC * NS

_mesh = functools.partial(
    plsc.VectorSubcoreMesh, core_axis_name="c", subcore_axis_name="s")


_EPT_A = N_EDGES // NS
_HALF = N_PAD // NC


@functools.partial(
    pl.kernel,
    out_type=jax.ShapeDtypeStruct((N_PAD,), jnp.int32),
    mesh=_mesh(),
    compiler_params=pltpu.CompilerParams(needs_layout_passes=False),
    scratch_types=[
        pltpu.VMEM((_EPT_A,), jnp.int32),
        pltpu.VMEM((N_PAD,), jnp.int32),
        pltpu.VMEM((_HALF // NS,), jnp.int32),
        pltpu.VMEM_SHARED((NS * N_PAD,), jnp.int32),
    ],
)
def _deg_kernel(dst_hbm, cnt_hbm, dst_v, acc_v, red_v, shared):
    cid = lax.axis_index("c")
    sid = lax.axis_index("s")
    zeros = jnp.zeros((L,), jnp.int32)

    def zero_body(i, _):
        acc_v[pl.ds(i * L, L)] = zeros

    lax.fori_loop(0, N_PAD // L, zero_body, None, unroll=8)

    pltpu.sync_copy(dst_hbm.at[pl.ds(sid * _EPT_A, _EPT_A)], dst_v)
    ones = jnp.ones((L,), jnp.int32)

    def body(i, _):
        idx = dst_v[pl.ds(i * L, L)]
        plsc.addupdate_scatter(acc_v, [idx], ones)

    lax.fori_loop(0, _EPT_A // L, body, None, unroll=4)

    pltpu.sync_copy(acc_v, shared.at[pl.ds(sid * N_PAD, N_PAD)])
    plsc.subcore_barrier()

    chunk = _HALF // NS
    base = cid * _HALF + sid * chunk

    def zero_red(i, _):
        red_v[pl.ds(i * L, L)] = zeros

    lax.fori_loop(0, chunk // L, zero_red, None, unroll=8)

    def red_body(t, _):
        pltpu.sync_copy(shared.at[pl.ds(t * N_PAD + base, chunk)],
                        dst_v.at[pl.ds(0, chunk)])

        def add_body(i, _):
            red_v[pl.ds(i * L, L)] = red_v[pl.ds(i * L, L)] + dst_v[pl.ds(i * L, L)]

        lax.fori_loop(0, chunk // L, add_body, None, unroll=8)

    lax.fori_loop(0, NS, red_body, None)
    pltpu.sync_copy(red_v, cnt_hbm.at[pl.ds(base, chunk)])


_EP_PAD = 327680
_EPC = _EP_PAD // NC
_EPT_B = _EPC // NS
_BB = 128
_NB = _EPT_B // _BB
_ROWS_PER_TILE = N_PAD // NS


@functools.partial(
    pl.kernel,
    out_type=jax.ShapeDtypeStruct((NC, N_PAD, FEAT), jnp.float32),
    mesh=_mesh(),
    compiler_params=pltpu.CompilerParams(needs_layout_passes=False),
    scratch_types=[
        pltpu.VMEM((_EPT_B + _BB,), jnp.int32),
        pltpu.VMEM((_BB,), jnp.int32),
        pltpu.VMEM((_BB,), jnp.int32),
        pltpu.VMEM((_BB,), jnp.int32),
        pltpu.VMEM((_BB,), jnp.int32),
        pltpu.VMEM((_BB, FEAT), jnp.float32),
        pltpu.VMEM((_BB, FEAT), jnp.float32),
        pltpu.VMEM_SHARED((N_PAD, FEAT), jnp.float32),
        pltpu.SemaphoreType.DMA,
        pltpu.SemaphoreType.DMA,
        pltpu.SemaphoreType.DMA,
    ],
)
def _msg_kernel(pk_hbm, p_hbm, out_hbm, pk_v, srcb0, srcb1, dstb0, dstb1,
                rows0, rows1, acc, sem_i, sem_g0, sem_g1):
    cid = lax.axis_index("c")
    sid = lax.axis_index("s")
    ebase = cid * _EPC + sid * _EPT_B

    h_pk = pltpu.async_copy(pk_hbm.at[pl.ds(ebase, _EPT_B)],
                            pk_v.at[pl.ds(0, _EPT_B)], sem_i)

    izeros = jnp.zeros((L,), jnp.int32)
    for j in range(_BB // L):
        pk_v[pl.ds(_EPT_B + j * L, L)] = izeros

    zeros = jnp.zeros((L,), jnp.float32)

    def zero_body(i, _):
        rows0[i // (FEAT // L), pl.ds((i % (FEAT // L)) * L, L)] = zeros

    lax.fori_loop(0, _BB * FEAT // L, zero_body, None, unroll=8)

    row0 = sid * _ROWS_PER_TILE
    for j in range(_ROWS_PER_TILE // _BB):
        pltpu.sync_copy(rows0, acc.at[pl.ds(row0 + j * _BB, _BB)])
    h_pk.wait()
    plsc.subcore_barrier()

    def unpack(b, sbuf, dbuf):
        def u(m, _):
            v = pk_v[pl.ds(b * _BB + m * L, L)]
            sbuf[pl.ds(m * L, L)] = v & 0xFFFF
            dbuf[pl.ds(m * L, L)] = v >> 16

        lax.fori_loop(0, _BB // L, u, None, unroll=8)

    def body(jo, _):
        b0 = jo * 2
        unpack(b0, srcb0, dstb0)
        pltpu.async_copy(p_hbm.at[srcb0], rows0, sem_g0).wait()
        pltpu.sync_copy(rows0, acc.at[dstb0], add=True)
        unpack(b0 + 1, srcb1, dstb1)
        pltpu.async_copy(p_hbm.at[srcb1], rows1, sem_g1).wait()
        pltpu.sync_copy(rows1, acc.at[dstb1], add=True)

    lax.fori_loop(0, _NB // 2, body, None)
    plsc.subcore_barrier()
    pltpu.sync_copy(acc.at[pl.ds(row0, _ROWS_PER_TILE)],
                    out_hbm.at[cid, pl.ds(row0, _ROWS_PER_TILE)])


_EPT_C = _EP_PAD // NS


@functools.partial(
    pl.kernel,
    out_type=jax.ShapeDtypeStruct((N_PAD,), jnp.float32),
    mesh=_mesh(),
    compiler_params=pltpu.CompilerParams(needs_layout_passes=False),
    scratch_types=[
        pltpu.VMEM((_EPT_C,), jnp.int32),
        pltpu.VMEM((N_PAD,), jnp.float32),
        pltpu.VMEM((N_PAD,), jnp.float32),
        pltpu.VMEM((_HALF // NS,), jnp.float32),
        pltpu.VMEM((_HALF // NS,), jnp.float32),
        pltpu.VMEM_SHARED((NS * N_PAD,), jnp.float32),
        pltpu.SemaphoreType.DMA,
    ],
)
def _scalar_kernel(pk_hbm, q_hbm, dinv_hbm, r_hbm, out_hbm,
                   pk_v, q_v, acc_v, red_v, aux_v, shared, sem_i):
    cid = lax.axis_index("c")
    sid = lax.axis_index("s")
    zeros = jnp.zeros((L,), jnp.float32)

    h_pk = pltpu.async_copy(pk_hbm.at[pl.ds(sid * _EPT_C, _EPT_C)], pk_v, sem_i)
    h_q = pltpu.async_copy(q_hbm, q_v, sem_i)

    def zero_body(i, _):
        acc_v[pl.ds(i * L, L)] = zeros

    lax.fori_loop(0, N_PAD // L, zero_body, None, unroll=8)
    h_pk.wait()
    h_q.wait()

    def body(i, _):
        v = pk_v[pl.ds(i * L, L)]
        s_idx = v & 0xFFFF
        d_idx = v >> 16
        vals = plsc.load_gather(q_v, [s_idx])
        plsc.addupdate_scatter(acc_v, [d_idx], vals)

    lax.fori_loop(0, _EPT_C // L, body, None, unroll=4)

    pltpu.sync_copy(acc_v, shared.at[pl.ds(sid * N_PAD, N_PAD)])
    plsc.subcore_barrier()

    chunk = _HALF // NS
    base = cid * _HALF + sid * chunk

    def zero_red(i, _):
        red_v[pl.ds(i * L, L)] = zeros

    lax.fori_loop(0, chunk // L, zero_red, None, unroll=8)

    def red_body(t, _):
        pltpu.sync_copy(shared.at[pl.ds(t * N_PAD + base, chunk)],
                        acc_v.at[pl.ds(0, chunk)])

        def add_body(i, _):
            red_v[pl.ds(i * L, L)] = red_v[pl.ds(i * L, L)] + acc_v[pl.ds(i * L, L)]

        lax.fori_loop(0, chunk // L, add_body, None, unroll=8)

    lax.fori_loop(0, NS, red_body, None)

    pltpu.sync_copy(dinv_hbm.at[pl.ds(base, chunk)], aux_v)

    def mul_body(i, _):
        red_v[pl.ds(i * L, L)] = red_v[pl.ds(i * L, L)] * aux_v[pl.ds(i * L, L)]

    lax.fori_loop(0, chunk // L, mul_body, None, unroll=8)
    pltpu.sync_copy(r_hbm.at[pl.ds(base, chunk)], aux_v)

    def add_r_body(i, _):
        red_v[pl.ds(i * L, L)] = red_v[pl.ds(i * L, L)] + aux_v[pl.ds(i * L, L)]

    lax.fori_loop(0, chunk // L, add_r_body, None, unroll=8)
    pltpu.sync_copy(red_v, out_hbm.at[pl.ds(base, chunk)])


_RB = 2048
_NRB = N_PAD // _RB


def _tc1_body(cnt_ref, x_ref, w1_ref, dinv_ref, p_ref):
    deg = cnt_ref[...].astype(jnp.float32) + 1.0
    dinv = lax.rsqrt(deg)
    dinv_ref[...] = dinv
    h = jnp.dot(x_ref[...], w1_ref[...], preferred_element_type=jnp.float32)
    p_ref[...] = dinv * h


def _tc1(cnt, x_pad, W1):
    return pl.pallas_call(
        _tc1_body,
        grid=(_NRB,),
        in_specs=[
            pl.BlockSpec((_RB, 1), lambda i: (i, 0)),
            pl.BlockSpec((_RB, FEAT), lambda i: (i, 0)),
            pl.BlockSpec((FEAT, FEAT), lambda i: (0, 0)),
        ],
        out_specs=[
            pl.BlockSpec((_RB, 1), lambda i: (i, 0)),
            pl.BlockSpec((_RB, FEAT), lambda i: (i, 0)),
        ],
        out_shape=[
            jax.ShapeDtypeStruct((N_PAD, 1), jnp.float32),
            jax.ShapeDtypeStruct((N_PAD, FEAT), jnp.float32),
        ],
    )(cnt, x_pad, W1)


def _tc2_body(s_ref, p_ref, dinv_ref, b1_ref, w2_ref, b2_ref, q_ref, r_ref):
    dinv = dinv_ref[...]
    s = s_ref[0] + s_ref[1] + p_ref[...]
    t = jnp.maximum(dinv * s + b1_ref[...], 0.0)
    z = jnp.dot(t, w2_ref[...], preferred_element_type=jnp.float32)
    q = dinv * z
    q_ref[...] = q
    r_ref[...] = dinv * q + b2_ref[0, 0]


def _tc2(s_parts, p, dinv, b1, W2, b2):
    return pl.pallas_call(
        _tc2_body,
        grid=(_NRB,),
        in_specs=[
            pl.BlockSpec((NC, _RB, FEAT), lambda i: (0, i, 0)),
            pl.BlockSpec((_RB, FEAT), lambda i: (i, 0)),
            pl.BlockSpec((_RB, 1), lambda i: (i, 0)),
            pl.BlockSpec((1, FEAT), lambda i: (0, 0)),
            pl.BlockSpec((FEAT, 1), lambda i: (0, 0)),
            pl.BlockSpec((1, 1), lambda i: (0, 0)),
        ],
        out_specs=[
            pl.BlockSpec((_RB, 1), lambda i: (i, 0)),
            pl.BlockSpec((_RB, 1), lambda i: (i, 0)),
        ],
        out_shape=[
            jax.ShapeDtypeStruct((N_PAD, 1), jnp.float32),
            jax.ShapeDtypeStruct((N_PAD, 1), jnp.float32),
        ],
    )(s_parts, p, dinv, b1, W2, b2)


def kernel(x, edge_index, W1, b1, W2, b2):
    edge_index = edge_index.astype(jnp.int32)
    src_idx = edge_index[0]
    dst_idx = edge_index[1]
    x_pad = jnp.pad(x, ((0, N_PAD - N_NODES), (0, 0)))

    src_pad = jnp.pad(src_idx, (0, _EP_PAD - N_EDGES))
    dst_pad = jnp.pad(dst_idx, (0, _EP_PAD - N_EDGES),
                      constant_values=N_PAD - 1)
    pk = jnp.bitwise_or(src_pad, jnp.left_shift(dst_pad, 16))

    cnt = _deg_kernel(dst_idx)
    dinv, p = _tc1(cnt.reshape(N_PAD, 1), x_pad, W1)
    s_parts = _msg_kernel(pk, p)
    q, r = _tc2(s_parts, p, dinv, b1.reshape(1, FEAT), W2, b2.reshape(1, 1))
    out = _scalar_kernel(pk, q.reshape(-1), dinv.reshape(-1), r.reshape(-1))
    return out[:N_NODES]


# speedup vs baseline: 1.2491x; 1.0757x over previous
import functools

import jax
import jax.numpy as jnp
from jax import lax
from jax.experimental import pallas as pl
from jax.experimental.pallas import tpu as pltpu
from jax.experimental.pallas import tpu_sc as plsc

N_NODES = 10000
N_PAD = 10240
N_EDGES = 320000
FEAT = 128

NC, NS, L = 2, 16, 16
NW = NC * NS

_mesh = functools.partial(
    plsc.VectorSubcoreMesh, core_axis_name="c", subcore_axis_name="s")


_EPT_A = N_EDGES // NS
_HALF = N_PAD // NC


@functools.partial(
    pl.kernel,
    out_type=jax.ShapeDtypeStruct((N_PAD,), jnp.int32),
    mesh=_mesh(),
    compiler_params=pltpu.CompilerParams(needs_layout_passes=False),
    scratch_types=[
        pltpu.VMEM((_EPT_A,), jnp.int32),
        pltpu.VMEM((N_PAD,), jnp.int32),
        pltpu.VMEM((_HALF // NS,), jnp.int32),
        pltpu.VMEM_SHARED((NS * N_PAD,), jnp.int32),
    ],
)
def _deg_kernel(dst_hbm, cnt_hbm, dst_v, acc_v, red_v, shared):
    cid = lax.axis_index("c")
    sid = lax.axis_index("s")
    zeros = jnp.zeros((L,), jnp.int32)

    def zero_body(i, _):
        acc_v[pl.ds(i * L, L)] = zeros

    lax.fori_loop(0, N_PAD // L, zero_body, None, unroll=8)

    pltpu.sync_copy(dst_hbm.at[pl.ds(sid * _EPT_A, _EPT_A)], dst_v)
    ones = jnp.ones((L,), jnp.int32)

    def body(i, _):
        idx = dst_v[pl.ds(i * L, L)]
        plsc.addupdate_scatter(acc_v, [idx], ones)

    lax.fori_loop(0, _EPT_A // L, body, None, unroll=4)

    pltpu.sync_copy(acc_v, shared.at[pl.ds(sid * N_PAD, N_PAD)])
    plsc.subcore_barrier()

    chunk = _HALF // NS
    base = cid * _HALF + sid * chunk

    def zero_red(i, _):
        red_v[pl.ds(i * L, L)] = zeros

    lax.fori_loop(0, chunk // L, zero_red, None, unroll=8)

    def red_body(t, _):
        pltpu.sync_copy(shared.at[pl.ds(t * N_PAD + base, chunk)],
                        dst_v.at[pl.ds(0, chunk)])

        def add_body(i, _):
            red_v[pl.ds(i * L, L)] = red_v[pl.ds(i * L, L)] + dst_v[pl.ds(i * L, L)]

        lax.fori_loop(0, chunk // L, add_body, None, unroll=8)

    lax.fori_loop(0, NS, red_body, None)
    pltpu.sync_copy(red_v, cnt_hbm.at[pl.ds(base, chunk)])


_EP_PAD = 327680
_EPC = _EP_PAD // NC
_EPT_B = _EPC // NS
_BB = 128
_NB = _EPT_B // _BB
_ROWS_PER_TILE = N_PAD // NS


@functools.partial(
    pl.kernel,
    out_type=jax.ShapeDtypeStruct((NC, N_PAD, FEAT), jnp.float32),
    mesh=_mesh(),
    compiler_params=pltpu.CompilerParams(needs_layout_passes=False),
    scratch_types=[
        pltpu.VMEM((_BB,), jnp.int32),
        pltpu.VMEM((_BB,), jnp.int32),
        pltpu.VMEM((_BB, FEAT), jnp.float32),
        pltpu.VMEM_SHARED((N_PAD, FEAT), jnp.float32),
        pltpu.SemaphoreType.DMA,
    ],
)
def _msg_kernel(src_hbm, dst_hbm, p_hbm, out_hbm, src_v, dst_v,
                rows_v, acc, sem_g):
    cid = lax.axis_index("c")
    sid = lax.axis_index("s")
    ebase = cid * _EPC + sid * _EPT_B

    zeros = jnp.zeros((L,), jnp.float32)

    def zero_body(i, _):
        rows_v[i // (FEAT // L), pl.ds((i % (FEAT // L)) * L, L)] = zeros

    lax.fori_loop(0, _BB * FEAT // L, zero_body, None, unroll=8)

    row0 = sid * _ROWS_PER_TILE
    for j in range(_ROWS_PER_TILE // _BB):
        pltpu.sync_copy(rows_v, acc.at[pl.ds(row0 + j * _BB, _BB)])
    plsc.subcore_barrier()

    def body(i, _):
        b = ebase + i * _BB
        pltpu.sync_copy(src_hbm.at[pl.ds(b, _BB)], src_v)
        pltpu.sync_copy(dst_hbm.at[pl.ds(b, _BB)], dst_v)
        pltpu.async_copy(p_hbm.at[src_v], rows_v, sem_g).wait()
        pltpu.sync_copy(rows_v, acc.at[dst_v], add=True)

    lax.fori_loop(0, _NB, body, None)
    plsc.subcore_barrier()
    pltpu.sync_copy(acc.at[pl.ds(row0, _ROWS_PER_TILE)],
                    out_hbm.at[cid, pl.ds(row0, _ROWS_PER_TILE)])


_EPT_C = _EP_PAD // NS


@functools.partial(
    pl.kernel,
    out_type=jax.ShapeDtypeStruct((N_PAD,), jnp.float32),
    mesh=_mesh(),
    compiler_params=pltpu.CompilerParams(needs_layout_passes=False),
    scratch_types=[
        pltpu.VMEM((_EPT_C,), jnp.int32),
        pltpu.VMEM((N_PAD,), jnp.float32),
        pltpu.VMEM((N_PAD,), jnp.float32),
        pltpu.VMEM((_HALF // NS,), jnp.float32),
        pltpu.VMEM((_HALF // NS,), jnp.float32),
        pltpu.VMEM_SHARED((NS * N_PAD,), jnp.float32),
        pltpu.SemaphoreType.DMA,
    ],
)
def _scalar_kernel(pk_hbm, q_hbm, dinv_hbm, r_hbm, out_hbm,
                   pk_v, q_v, acc_v, red_v, aux_v, shared, sem_i):
    cid = lax.axis_index("c")
    sid = lax.axis_index("s")
    zeros = jnp.zeros((L,), jnp.float32)

    h_pk = pltpu.async_copy(pk_hbm.at[pl.ds(sid * _EPT_C, _EPT_C)], pk_v, sem_i)
    h_q = pltpu.async_copy(q_hbm, q_v, sem_i)

    def zero_body(i, _):
        acc_v[pl.ds(i * L, L)] = zeros

    lax.fori_loop(0, N_PAD // L, zero_body, None, unroll=8)
    h_pk.wait()
    h_q.wait()

    def body(i, _):
        v = pk_v[pl.ds(i * L, L)]
        s_idx = v & 0xFFFF
        d_idx = v >> 16
        vals = plsc.load_gather(q_v, [s_idx])
        plsc.addupdate_scatter(acc_v, [d_idx], vals)

    lax.fori_loop(0, _EPT_C // L, body, None, unroll=4)

    pltpu.sync_copy(acc_v, shared.at[pl.ds(sid * N_PAD, N_PAD)])
    plsc.subcore_barrier()

    chunk = _HALF // NS
    base = cid * _HALF + sid * chunk

    def zero_red(i, _):
        red_v[pl.ds(i * L, L)] = zeros

    lax.fori_loop(0, chunk // L, zero_red, None, unroll=8)

    def red_body(t, _):
        pltpu.sync_copy(shared.at[pl.ds(t * N_PAD + base, chunk)],
                        acc_v.at[pl.ds(0, chunk)])

        def add_body(i, _):
            red_v[pl.ds(i * L, L)] = red_v[pl.ds(i * L, L)] + acc_v[pl.ds(i * L, L)]

        lax.fori_loop(0, chunk // L, add_body, None, unroll=8)

    lax.fori_loop(0, NS, red_body, None)

    pltpu.sync_copy(dinv_hbm.at[pl.ds(base, chunk)], aux_v)

    def mul_body(i, _):
        red_v[pl.ds(i * L, L)] = red_v[pl.ds(i * L, L)] * aux_v[pl.ds(i * L, L)]

    lax.fori_loop(0, chunk // L, mul_body, None, unroll=8)
    pltpu.sync_copy(r_hbm.at[pl.ds(base, chunk)], aux_v)

    def add_r_body(i, _):
        red_v[pl.ds(i * L, L)] = red_v[pl.ds(i * L, L)] + aux_v[pl.ds(i * L, L)]

    lax.fori_loop(0, chunk // L, add_r_body, None, unroll=8)
    pltpu.sync_copy(red_v, out_hbm.at[pl.ds(base, chunk)])


_RB = 2048
_NRB = N_PAD // _RB


def _tc1_body(cnt_ref, x_ref, w1_ref, dinv_ref, p_ref):
    deg = cnt_ref[...].astype(jnp.float32) + 1.0
    dinv = lax.rsqrt(deg)
    dinv_ref[...] = dinv
    h = jnp.dot(x_ref[...], w1_ref[...], preferred_element_type=jnp.float32)
    p_ref[...] = dinv * h


def _tc1(cnt, x_pad, W1):
    return pl.pallas_call(
        _tc1_body,
        grid=(_NRB,),
        in_specs=[
            pl.BlockSpec((_RB, 1), lambda i: (i, 0)),
            pl.BlockSpec((_RB, FEAT), lambda i: (i, 0)),
            pl.BlockSpec((FEAT, FEAT), lambda i: (0, 0)),
        ],
        out_specs=[
            pl.BlockSpec((_RB, 1), lambda i: (i, 0)),
            pl.BlockSpec((_RB, FEAT), lambda i: (i, 0)),
        ],
        out_shape=[
            jax.ShapeDtypeStruct((N_PAD, 1), jnp.float32),
            jax.ShapeDtypeStruct((N_PAD, FEAT), jnp.float32),
        ],
    )(cnt, x_pad, W1)


def _tc2_body(s_ref, p_ref, dinv_ref, b1_ref, w2_ref, b2_ref, q_ref, r_ref):
    dinv = dinv_ref[...]
    s = s_ref[0] + s_ref[1] + p_ref[...]
    t = jnp.maximum(dinv * s + b1_ref[...], 0.0)
    z = jnp.dot(t, w2_ref[...], preferred_element_type=jnp.float32)
    q = dinv * z
    q_ref[...] = q
    r_ref[...] = dinv * q + b2_ref[0, 0]


def _tc2(s_parts, p, dinv, b1, W2, b2):
    return pl.pallas_call(
        _tc2_body,
        grid=(_NRB,),
        in_specs=[
            pl.BlockSpec((NC, _RB, FEAT), lambda i: (0, i, 0)),
            pl.BlockSpec((_RB, FEAT), lambda i: (i, 0)),
            pl.BlockSpec((_RB, 1), lambda i: (i, 0)),
            pl.BlockSpec((1, FEAT), lambda i: (0, 0)),
            pl.BlockSpec((FEAT, 1), lambda i: (0, 0)),
            pl.BlockSpec((1, 1), lambda i: (0, 0)),
        ],
        out_specs=[
            pl.BlockSpec((_RB, 1), lambda i: (i, 0)),
            pl.BlockSpec((_RB, 1), lambda i: (i, 0)),
        ],
        out_shape=[
            jax.ShapeDtypeStruct((N_PAD, 1), jnp.float32),
            jax.ShapeDtypeStruct((N_PAD, 1), jnp.float32),
        ],
    )(s_parts, p, dinv, b1, W2, b2)


def kernel(x, edge_index, W1, b1, W2, b2):
    edge_index = edge_index.astype(jnp.int32)
    src_idx = edge_index[0]
    dst_idx = edge_index[1]
    x_pad = jnp.pad(x, ((0, N_PAD - N_NODES), (0, 0)))

    src_pad = jnp.pad(src_idx, (0, _EP_PAD - N_EDGES))
    dst_pad = jnp.pad(dst_idx, (0, _EP_PAD - N_EDGES),
                      constant_values=N_PAD - 1)
    pk = jnp.bitwise_or(src_pad, jnp.left_shift(dst_pad, 16))

    cnt = _deg_kernel(dst_idx)
    dinv, p = _tc1(cnt.reshape(N_PAD, 1), x_pad, W1)
    s_parts = _msg_kernel(src_pad, dst_pad, p)
    q, r = _tc2(s_parts, p, dinv, b1.reshape(1, FEAT), W2, b2.reshape(1, 1))
    out = _scalar_kernel(pk, q.reshape(-1), dinv.reshape(-1), r.reshape(-1))
    return out[:N_NODES]


# speedup vs baseline: 2.7112x; 2.1706x over previous
import functools

import jax
import jax.numpy as jnp
from jax import lax
from jax.experimental import pallas as pl
from jax.experimental.pallas import tpu as pltpu
from jax.experimental.pallas import tpu_sc as plsc

N_NODES = 10000
N_PAD = 10240
N_EDGES = 320000
FEAT = 128

NC, NS, L = 2, 16, 16
NW = NC * NS

_mesh = functools.partial(
    plsc.VectorSubcoreMesh, core_axis_name="c", subcore_axis_name="s")


_EPT_A = N_EDGES // NS
_HALF = N_PAD // NC


@functools.partial(
    pl.kernel,
    out_type=jax.ShapeDtypeStruct((N_PAD,), jnp.int32),
    mesh=_mesh(),
    compiler_params=pltpu.CompilerParams(needs_layout_passes=False),
    scratch_types=[
        pltpu.VMEM((_EPT_A,), jnp.int32),
        pltpu.VMEM((N_PAD,), jnp.int32),
        pltpu.VMEM((_HALF // NS,), jnp.int32),
        pltpu.VMEM_SHARED((NS * N_PAD,), jnp.int32),
    ],
)
def _deg_kernel(dst_hbm, cnt_hbm, dst_v, acc_v, red_v, shared):
    cid = lax.axis_index("c")
    sid = lax.axis_index("s")
    zeros = jnp.zeros((L,), jnp.int32)

    def zero_body(i, _):
        acc_v[pl.ds(i * L, L)] = zeros

    lax.fori_loop(0, N_PAD // L, zero_body, None, unroll=8)

    pltpu.sync_copy(dst_hbm.at[pl.ds(sid * _EPT_A, _EPT_A)], dst_v)
    ones = jnp.ones((L,), jnp.int32)

    def body(i, _):
        idx = dst_v[pl.ds(i * L, L)]
        plsc.addupdate_scatter(acc_v, [idx], ones)

    lax.fori_loop(0, _EPT_A // L, body, None, unroll=4)

    pltpu.sync_copy(acc_v, shared.at[pl.ds(sid * N_PAD, N_PAD)])
    plsc.subcore_barrier()

    chunk = _HALF // NS
    base = cid * _HALF + sid * chunk

    def zero_red(i, _):
        red_v[pl.ds(i * L, L)] = zeros

    lax.fori_loop(0, chunk // L, zero_red, None, unroll=8)

    def red_body(t, _):
        pltpu.sync_copy(shared.at[pl.ds(t * N_PAD + base, chunk)],
                        dst_v.at[pl.ds(0, chunk)])

        def add_body(i, _):
            red_v[pl.ds(i * L, L)] = red_v[pl.ds(i * L, L)] + dst_v[pl.ds(i * L, L)]

        lax.fori_loop(0, chunk // L, add_body, None, unroll=8)

    lax.fori_loop(0, NS, red_body, None)
    pltpu.sync_copy(red_v, cnt_hbm.at[pl.ds(base, chunk)])


_EPC = N_EDGES // NC
_EPT_B = _EPC // NS
_BB = 80
_NB = _EPT_B // _BB
_ROWS_PER_TILE = N_PAD // NS


@functools.partial(
    pl.kernel,
    out_type=jax.ShapeDtypeStruct((NC, N_PAD, FEAT), jnp.float32),
    mesh=_mesh(),
    compiler_params=pltpu.CompilerParams(needs_layout_passes=False),
    scratch_types=[
        pltpu.VMEM((_EPT_B,), jnp.int32),
        pltpu.VMEM((_BB,), jnp.int32),
        pltpu.VMEM((_BB,), jnp.int32),
        pltpu.VMEM((_BB, FEAT), jnp.float32),
        pltpu.VMEM_SHARED((N_PAD, FEAT), jnp.float32),
        pltpu.SemaphoreType.DMA,
        pltpu.SemaphoreType.DMA,
        pltpu.SemaphoreType.DMA,
    ],
)
def _msg_kernel(src_hbm, dst_hbm, p_hbm, out_hbm, src_v, dstb0, dstb1,
                rows_v, acc, sem_i, sem_d, sem_g):
    cid = lax.axis_index("c")
    sid = lax.axis_index("s")
    ebase = cid * _EPC + sid * _EPT_B

    h_src = pltpu.async_copy(src_hbm.at[pl.ds(ebase, _EPT_B)], src_v, sem_i)

    zeros = jnp.zeros((L,), jnp.float32)

    def zero_body(i, _):
        rows_v[i // (FEAT // L), pl.ds((i % (FEAT // L)) * L, L)] = zeros

    lax.fori_loop(0, _BB * FEAT // L, zero_body, None, unroll=8)

    row0 = sid * _ROWS_PER_TILE
    for j in range(_ROWS_PER_TILE // _BB):
        pltpu.sync_copy(rows_v, acc.at[pl.ds(row0 + j * _BB, _BB)])
    h_src.wait()
    plsc.subcore_barrier()

    pltpu.async_copy(dst_hbm.at[pl.ds(ebase, _BB)], dstb0, sem_d)

    def half(b, dst_cur, dst_nxt):
        pltpu.async_copy(dst_hbm.at[pl.ds(ebase + (b + 1) * _BB, _BB)],
                         dst_nxt, sem_d)
        pltpu.async_copy(p_hbm.at[src_v.at[pl.ds(b * _BB, _BB)]],
                         rows_v, sem_g).wait()
        pltpu.make_async_copy(dst_hbm.at[pl.ds(ebase + b * _BB, _BB)],
                              dst_cur, sem_d).wait()
        pltpu.sync_copy(rows_v, acc.at[dst_cur], add=True)

    def body(jo, _):
        half(jo * 2, dstb0, dstb1)
        half(jo * 2 + 1, dstb1, dstb0)

    lax.fori_loop(0, (_NB - 1) // 2, body, None)

    b_last = _NB - 1
    pltpu.async_copy(p_hbm.at[src_v.at[pl.ds(b_last * _BB, _BB)]],
                     rows_v, sem_g).wait()
    pltpu.make_async_copy(dst_hbm.at[pl.ds(ebase + b_last * _BB, _BB)],
                          dstb0, sem_d).wait()
    pltpu.sync_copy(rows_v, acc.at[dstb0], add=True)

    plsc.subcore_barrier()
    pltpu.sync_copy(acc.at[pl.ds(row0, _ROWS_PER_TILE)],
                    out_hbm.at[cid, pl.ds(row0, _ROWS_PER_TILE)])


_EPT_C = N_EDGES // NS


@functools.partial(
    pl.kernel,
    out_type=jax.ShapeDtypeStruct((N_PAD,), jnp.float32),
    mesh=_mesh(),
    compiler_params=pltpu.CompilerParams(needs_layout_passes=False),
    scratch_types=[
        pltpu.VMEM((_EPT_C,), jnp.int32),
        pltpu.VMEM((N_PAD,), jnp.float32),
        pltpu.VMEM((N_PAD,), jnp.float32),
        pltpu.VMEM((_HALF // NS,), jnp.float32),
        pltpu.VMEM((_HALF // NS,), jnp.float32),
        pltpu.VMEM_SHARED((NS * N_PAD,), jnp.float32),
        pltpu.SemaphoreType.DMA,
    ],
)
def _scalar_kernel(pk_hbm, q_hbm, dinv_hbm, r_hbm, out_hbm,
                   pk_v, q_v, acc_v, red_v, aux_v, shared, sem_i):
    cid = lax.axis_index("c")
    sid = lax.axis_index("s")
    zeros = jnp.zeros((L,), jnp.float32)

    h_pk = pltpu.async_copy(pk_hbm.at[pl.ds(sid * _EPT_C, _EPT_C)], pk_v, sem_i)
    h_q = pltpu.async_copy(q_hbm, q_v, sem_i)

    def zero_body(i, _):
        acc_v[pl.ds(i * L, L)] = zeros

    lax.fori_loop(0, N_PAD // L, zero_body, None, unroll=8)
    h_pk.wait()
    h_q.wait()

    def body(i, _):
        v = pk_v[pl.ds(i * L, L)]
        s_idx = v & 0xFFFF
        d_idx = v >> 16
        vals = plsc.load_gather(q_v, [s_idx])
        plsc.addupdate_scatter(acc_v, [d_idx], vals)

    lax.fori_loop(0, _EPT_C // L, body, None, unroll=4)

    pltpu.sync_copy(acc_v, shared.at[pl.ds(sid * N_PAD, N_PAD)])
    plsc.subcore_barrier()

    chunk = _HALF // NS
    base = cid * _HALF + sid * chunk

    def zero_red(i, _):
        red_v[pl.ds(i * L, L)] = zeros

    lax.fori_loop(0, chunk // L, zero_red, None, unroll=8)

    def red_body(t, _):
        pltpu.sync_copy(shared.at[pl.ds(t * N_PAD + base, chunk)],
                        acc_v.at[pl.ds(0, chunk)])

        def add_body(i, _):
            red_v[pl.ds(i * L, L)] = red_v[pl.ds(i * L, L)] + acc_v[pl.ds(i * L, L)]

        lax.fori_loop(0, chunk // L, add_body, None, unroll=8)

    lax.fori_loop(0, NS, red_body, None)

    pltpu.sync_copy(dinv_hbm.at[pl.ds(base, chunk)], aux_v)

    def mul_body(i, _):
        red_v[pl.ds(i * L, L)] = red_v[pl.ds(i * L, L)] * aux_v[pl.ds(i * L, L)]

    lax.fori_loop(0, chunk // L, mul_body, None, unroll=8)
    pltpu.sync_copy(r_hbm.at[pl.ds(base, chunk)], aux_v)

    def add_r_body(i, _):
        red_v[pl.ds(i * L, L)] = red_v[pl.ds(i * L, L)] + aux_v[pl.ds(i * L, L)]

    lax.fori_loop(0, chunk // L, add_r_body, None, unroll=8)
    pltpu.sync_copy(red_v, out_hbm.at[pl.ds(base, chunk)])


_RB = 2048
_NRB = N_PAD // _RB


def _tc1_body(cnt_ref, x_ref, w1_ref, dinv_ref, p_ref):
    deg = cnt_ref[...].astype(jnp.float32) + 1.0
    dinv = lax.rsqrt(deg)
    dinv_ref[...] = dinv
    h = jnp.dot(x_ref[...], w1_ref[...], preferred_element_type=jnp.float32)
    p_ref[...] = dinv * h


def _tc1(cnt, x_pad, W1):
    return pl.pallas_call(
        _tc1_body,
        grid=(_NRB,),
        in_specs=[
            pl.BlockSpec((_RB, 1), lambda i: (i, 0)),
            pl.BlockSpec((_RB, FEAT), lambda i: (i, 0)),
            pl.BlockSpec((FEAT, FEAT), lambda i: (0, 0)),
        ],
        out_specs=[
            pl.BlockSpec((_RB, 1), lambda i: (i, 0)),
            pl.BlockSpec((_RB, FEAT), lambda i: (i, 0)),
        ],
        out_shape=[
            jax.ShapeDtypeStruct((N_PAD, 1), jnp.float32),
            jax.ShapeDtypeStruct((N_PAD, FEAT), jnp.float32),
        ],
    )(cnt, x_pad, W1)


def _tc2_body(s_ref, p_ref, dinv_ref, b1_ref, w2_ref, b2_ref, q_ref, r_ref):
    dinv = dinv_ref[...]
    s = s_ref[0] + s_ref[1] + p_ref[...]
    t = jnp.maximum(dinv * s + b1_ref[...], 0.0)
    z = jnp.dot(t, w2_ref[...], preferred_element_type=jnp.float32)
    q = dinv * z
    q_ref[...] = q
    r_ref[...] = dinv * q + b2_ref[0, 0]


def _tc2(s_parts, p, dinv, b1, W2, b2):
    return pl.pallas_call(
        _tc2_body,
        grid=(_NRB,),
        in_specs=[
            pl.BlockSpec((NC, _RB, FEAT), lambda i: (0, i, 0)),
            pl.BlockSpec((_RB, FEAT), lambda i: (i, 0)),
            pl.BlockSpec((_RB, 1), lambda i: (i, 0)),
            pl.BlockSpec((1, FEAT), lambda i: (0, 0)),
            pl.BlockSpec((FEAT, 1), lambda i: (0, 0)),
            pl.BlockSpec((1, 1), lambda i: (0, 0)),
        ],
        out_specs=[
            pl.BlockSpec((_RB, 1), lambda i: (i, 0)),
            pl.BlockSpec((_RB, 1), lambda i: (i, 0)),
        ],
        out_shape=[
            jax.ShapeDtypeStruct((N_PAD, 1), jnp.float32),
            jax.ShapeDtypeStruct((N_PAD, 1), jnp.float32),
        ],
    )(s_parts, p, dinv, b1, W2, b2)


def kernel(x, edge_index, W1, b1, W2, b2):
    edge_index = edge_index.astype(jnp.int32)
    src_idx = edge_index[0]
    dst_idx = edge_index[1]
    x_pad = jnp.pad(x, ((0, N_PAD - N_NODES), (0, 0)))

    pk = jnp.bitwise_or(src_idx, jnp.left_shift(dst_idx, 16))

    cnt = _deg_kernel(dst_idx)
    dinv, p = _tc1(cnt.reshape(N_PAD, 1), x_pad, W1)
    s_parts = _msg_kernel(src_idx, dst_idx, p)
    q, r = _tc2(s_parts, p, dinv, b1.reshape(1, FEAT), W2, b2.reshape(1, 1))
    out = _scalar_kernel(pk, q.reshape(-1), dinv.reshape(-1), r.reshape(-1))
    return out[:N_NODES]


# speedup vs baseline: 3.8337x; 1.4140x over previous
import functools

import jax
import jax.numpy as jnp
from jax import lax
from jax.experimental import pallas as pl
from jax.experimental.pallas import tpu as pltpu
from jax.experimental.pallas import tpu_sc as plsc

N_NODES = 10000
N_PAD = 10240
N_EDGES = 320000
FEAT = 128

NC, NS, L = 2, 16, 16
NW = NC * NS

_mesh = functools.partial(
    plsc.VectorSubcoreMesh, core_axis_name="c", subcore_axis_name="s")


_EPT_A = N_EDGES // NS
_HALF = N_PAD // NC


@functools.partial(
    pl.kernel,
    out_type=jax.ShapeDtypeStruct((N_PAD,), jnp.int32),
    mesh=_mesh(),
    compiler_params=pltpu.CompilerParams(needs_layout_passes=False),
    scratch_types=[
        pltpu.VMEM((_EPT_A,), jnp.int32),
        pltpu.VMEM((N_PAD,), jnp.int32),
        pltpu.VMEM((_HALF // NS,), jnp.int32),
        pltpu.VMEM_SHARED((NS * N_PAD,), jnp.int32),
    ],
)
def _deg_kernel(dst_hbm, cnt_hbm, dst_v, acc_v, red_v, shared):
    cid = lax.axis_index("c")
    sid = lax.axis_index("s")
    zeros = jnp.zeros((L,), jnp.int32)

    def zero_body(i, _):
        acc_v[pl.ds(i * L, L)] = zeros

    lax.fori_loop(0, N_PAD // L, zero_body, None, unroll=8)

    pltpu.sync_copy(dst_hbm.at[pl.ds(sid * _EPT_A, _EPT_A)], dst_v)
    ones = jnp.ones((L,), jnp.int32)

    def body(i, _):
        idx = dst_v[pl.ds(i * L, L)]
        plsc.addupdate_scatter(acc_v, [idx], ones)

    lax.fori_loop(0, _EPT_A // L, body, None, unroll=4)

    pltpu.sync_copy(acc_v, shared.at[pl.ds(sid * N_PAD, N_PAD)])
    plsc.subcore_barrier()

    chunk = _HALF // NS
    base = cid * _HALF + sid * chunk

    def zero_red(i, _):
        red_v[pl.ds(i * L, L)] = zeros

    lax.fori_loop(0, chunk // L, zero_red, None, unroll=8)

    def red_body(t, _):
        pltpu.sync_copy(shared.at[pl.ds(t * N_PAD + base, chunk)],
                        dst_v.at[pl.ds(0, chunk)])

        def add_body(i, _):
            red_v[pl.ds(i * L, L)] = red_v[pl.ds(i * L, L)] + dst_v[pl.ds(i * L, L)]

        lax.fori_loop(0, chunk // L, add_body, None, unroll=8)

    lax.fori_loop(0, NS, red_body, None)
    pltpu.sync_copy(red_v, cnt_hbm.at[pl.ds(base, chunk)])


_EPC = N_EDGES // NC
_EPT_B = _EPC // NS
_BB = 80
_NB = _EPT_B // _BB
_ROWS_PER_TILE = N_PAD // NS


@functools.partial(
    pl.kernel,
    out_type=jax.ShapeDtypeStruct((NC, N_PAD, FEAT), jnp.float32),
    mesh=_mesh(),
    compiler_params=pltpu.CompilerParams(needs_layout_passes=False),
    scratch_types=[
        pltpu.VMEM((_EPT_B,), jnp.int32),
        pltpu.VMEM((_BB,), jnp.int32),
        pltpu.VMEM((_BB,), jnp.int32),
        pltpu.VMEM((_BB, FEAT), jnp.float32),
        pltpu.VMEM((_BB, FEAT), jnp.float32),
        pltpu.VMEM_SHARED((N_PAD, FEAT), jnp.float32),
        pltpu.SemaphoreType.DMA,
        pltpu.SemaphoreType.DMA,
        pltpu.SemaphoreType.DMA,
        pltpu.SemaphoreType.DMA,
    ],
)
def _msg_kernel(src_hbm, dst_hbm, p_hbm, out_hbm, src_v, dstb0, dstb1,
                rows0, rows1, acc, sem_i, sem_d, sem_g0, sem_g1):
    cid = lax.axis_index("c")
    sid = lax.axis_index("s")
    ebase = cid * _EPC + sid * _EPT_B

    h_src = pltpu.async_copy(src_hbm.at[pl.ds(ebase, _EPT_B)], src_v, sem_i)

    zeros = jnp.zeros((L,), jnp.float32)

    def zero_body(i, _):
        rows0[i // (FEAT // L), pl.ds((i % (FEAT // L)) * L, L)] = zeros

    lax.fori_loop(0, _BB * FEAT // L, zero_body, None, unroll=8)

    row0 = sid * _ROWS_PER_TILE
    for j in range(_ROWS_PER_TILE // _BB):
        pltpu.sync_copy(rows0, acc.at[pl.ds(row0 + j * _BB, _BB)])
    h_src.wait()
    plsc.subcore_barrier()

    def gather(b, rows, sem):
        return pltpu.async_copy(
            p_hbm.at[src_v.at[pl.ds(b * _BB, _BB)]], rows, sem)

    def gather_wait(b, rows, sem):
        pltpu.make_async_copy(
            p_hbm.at[src_v.at[pl.ds(b * _BB, _BB)]], rows, sem).wait()

    pltpu.async_copy(dst_hbm.at[pl.ds(ebase, _BB)], dstb0, sem_d)
    gather(0, rows0, sem_g0)

    def half(b, dst_cur, dst_nxt, rows_cur, sem_cur, rows_nxt, sem_nxt):
        pltpu.async_copy(dst_hbm.at[pl.ds(ebase + (b + 1) * _BB, _BB)],
                         dst_nxt, sem_d)
        gather(b + 1, rows_nxt, sem_nxt)
        gather_wait(b, rows_cur, sem_cur)
        pltpu.make_async_copy(dst_hbm.at[pl.ds(ebase + b * _BB, _BB)],
                              dst_cur, sem_d).wait()
        pltpu.sync_copy(rows_cur, acc.at[dst_cur], add=True)

    def body(jo, _):
        half(jo * 2, dstb0, dstb1, rows0, sem_g0, rows1, sem_g1)
        half(jo * 2 + 1, dstb1, dstb0, rows1, sem_g1, rows0, sem_g0)

    lax.fori_loop(0, (_NB - 1) // 2, body, None)

    b_last = _NB - 1
    gather_wait(b_last, rows0, sem_g0)
    pltpu.make_async_copy(dst_hbm.at[pl.ds(ebase + b_last * _BB, _BB)],
                          dstb0, sem_d).wait()
    pltpu.sync_copy(rows0, acc.at[dstb0], add=True)

    plsc.subcore_barrier()
    pltpu.sync_copy(acc.at[pl.ds(row0, _ROWS_PER_TILE)],
                    out_hbm.at[cid, pl.ds(row0, _ROWS_PER_TILE)])


_EPT_C = N_EDGES // NS


@functools.partial(
    pl.kernel,
    out_type=jax.ShapeDtypeStruct((N_PAD,), jnp.float32),
    mesh=_mesh(),
    compiler_params=pltpu.CompilerParams(needs_layout_passes=False),
    scratch_types=[
        pltpu.VMEM((_EPT_C,), jnp.int32),
        pltpu.VMEM((N_PAD,), jnp.float32),
        pltpu.VMEM((N_PAD,), jnp.float32),
        pltpu.VMEM((_HALF // NS,), jnp.float32),
        pltpu.VMEM((_HALF // NS,), jnp.float32),
        pltpu.VMEM_SHARED((NS * N_PAD,), jnp.float32),
        pltpu.SemaphoreType.DMA,
    ],
)
def _scalar_kernel(pk_hbm, q_hbm, dinv_hbm, r_hbm, out_hbm,
                   pk_v, q_v, acc_v, red_v, aux_v, shared, sem_i):
    cid = lax.axis_index("c")
    sid = lax.axis_index("s")
    zeros = jnp.zeros((L,), jnp.float32)

    h_pk = pltpu.async_copy(pk_hbm.at[pl.ds(sid * _EPT_C, _EPT_C)], pk_v, sem_i)
    h_q = pltpu.async_copy(q_hbm, q_v, sem_i)

    def zero_body(i, _):
        acc_v[pl.ds(i * L, L)] = zeros

    lax.fori_loop(0, N_PAD // L, zero_body, None, unroll=8)
    h_pk.wait()
    h_q.wait()

    def body(i, _):
        v = pk_v[pl.ds(i * L, L)]
        s_idx = v & 0xFFFF
        d_idx = v >> 16
        vals = plsc.load_gather(q_v, [s_idx])
        plsc.addupdate_scatter(acc_v, [d_idx], vals)

    lax.fori_loop(0, _EPT_C // L, body, None, unroll=4)

    pltpu.sync_copy(acc_v, shared.at[pl.ds(sid * N_PAD, N_PAD)])
    plsc.subcore_barrier()

    chunk = _HALF // NS
    base = cid * _HALF + sid * chunk

    def zero_red(i, _):
        red_v[pl.ds(i * L, L)] = zeros

    lax.fori_loop(0, chunk // L, zero_red, None, unroll=8)

    def red_body(t, _):
        pltpu.sync_copy(shared.at[pl.ds(t * N_PAD + base, chunk)],
                        acc_v.at[pl.ds(0, chunk)])

        def add_body(i, _):
            red_v[pl.ds(i * L, L)] = red_v[pl.ds(i * L, L)] + acc_v[pl.ds(i * L, L)]

        lax.fori_loop(0, chunk // L, add_body, None, unroll=8)

    lax.fori_loop(0, NS, red_body, None)

    pltpu.sync_copy(dinv_hbm.at[pl.ds(base, chunk)], aux_v)

    def mul_body(i, _):
        red_v[pl.ds(i * L, L)] = red_v[pl.ds(i * L, L)] * aux_v[pl.ds(i * L, L)]

    lax.fori_loop(0, chunk // L, mul_body, None, unroll=8)
    pltpu.sync_copy(r_hbm.at[pl.ds(base, chunk)], aux_v)

    def add_r_body(i, _):
        red_v[pl.ds(i * L, L)] = red_v[pl.ds(i * L, L)] + aux_v[pl.ds(i * L, L)]

    lax.fori_loop(0, chunk // L, add_r_body, None, unroll=8)
    pltpu.sync_copy(red_v, out_hbm.at[pl.ds(base, chunk)])


_RB = 2048
_NRB = N_PAD // _RB


def _tc1_body(cnt_ref, x_ref, w1_ref, dinv_ref, p_ref):
    deg = cnt_ref[...].astype(jnp.float32) + 1.0
    dinv = lax.rsqrt(deg)
    dinv_ref[...] = dinv
    h = jnp.dot(x_ref[...], w1_ref[...], preferred_element_type=jnp.float32)
    p_ref[...] = dinv * h


def _tc1(cnt, x_pad, W1):
    return pl.pallas_call(
        _tc1_body,
        grid=(_NRB,),
        in_specs=[
            pl.BlockSpec((_RB, 1), lambda i: (i, 0)),
            pl.BlockSpec((_RB, FEAT), lambda i: (i, 0)),
            pl.BlockSpec((FEAT, FEAT), lambda i: (0, 0)),
        ],
        out_specs=[
            pl.BlockSpec((_RB, 1), lambda i: (i, 0)),
            pl.BlockSpec((_RB, FEAT), lambda i: (i, 0)),
        ],
        out_shape=[
            jax.ShapeDtypeStruct((N_PAD, 1), jnp.float32),
            jax.ShapeDtypeStruct((N_PAD, FEAT), jnp.float32),
        ],
    )(cnt, x_pad, W1)


def _tc2_body(s_ref, p_ref, dinv_ref, b1_ref, w2_ref, b2_ref, q_ref, r_ref):
    dinv = dinv_ref[...]
    s = s_ref[0] + s_ref[1] + p_ref[...]
    t = jnp.maximum(dinv * s + b1_ref[...], 0.0)
    z = jnp.dot(t, w2_ref[...], preferred_element_type=jnp.float32)
    q = dinv * z
    q_ref[...] = q
    r_ref[...] = dinv * q + b2_ref[0, 0]


def _tc2(s_parts, p, dinv, b1, W2, b2):
    return pl.pallas_call(
        _tc2_body,
        grid=(_NRB,),
        in_specs=[
            pl.BlockSpec((NC, _RB, FEAT), lambda i: (0, i, 0)),
            pl.BlockSpec((_RB, FEAT), lambda i: (i, 0)),
            pl.BlockSpec((_RB, 1), lambda i: (i, 0)),
            pl.BlockSpec((1, FEAT), lambda i: (0, 0)),
            pl.BlockSpec((FEAT, 1), lambda i: (0, 0)),
            pl.BlockSpec((1, 1), lambda i: (0, 0)),
        ],
        out_specs=[
            pl.BlockSpec((_RB, 1), lambda i: (i, 0)),
            pl.BlockSpec((_RB, 1), lambda i: (i, 0)),
        ],
        out_shape=[
            jax.ShapeDtypeStruct((N_PAD, 1), jnp.float32),
            jax.ShapeDtypeStruct((N_PAD, 1), jnp.float32),
        ],
    )(s_parts, p, dinv, b1, W2, b2)


def kernel(x, edge_index, W1, b1, W2, b2):
    edge_index = edge_index.astype(jnp.int32)
    src_idx = edge_index[0]
    dst_idx = edge_index[1]
    x_pad = jnp.pad(x, ((0, N_PAD - N_NODES), (0, 0)))

    pk = jnp.bitwise_or(src_idx, jnp.left_shift(dst_idx, 16))

    cnt = _deg_kernel(dst_idx)
    dinv, p = _tc1(cnt.reshape(N_PAD, 1), x_pad, W1)
    s_parts = _msg_kernel(src_idx, dst_idx, p)
    q, r = _tc2(s_parts, p, dinv, b1.reshape(1, FEAT), W2, b2.reshape(1, 1))
    out = _scalar_kernel(pk, q.reshape(-1), dinv.reshape(-1), r.reshape(-1))
    return out[:N_NODES]


# speedup vs baseline: 3.8908x; 1.0149x over previous
import functools

import jax
import jax.numpy as jnp
from jax import lax
from jax.experimental import pallas as pl
from jax.experimental.pallas import tpu as pltpu
from jax.experimental.pallas import tpu_sc as plsc

N_NODES = 10000
N_PAD = 10240
N_EDGES = 320000
FEAT = 128

NC, NS, L = 2, 16, 16
NW = NC * NS

_mesh = functools.partial(
    plsc.VectorSubcoreMesh, core_axis_name="c", subcore_axis_name="s")


_EPT_A = N_EDGES // NS
_HALF = N_PAD // NC


@functools.partial(
    pl.kernel,
    out_type=jax.ShapeDtypeStruct((N_PAD,), jnp.int32),
    mesh=_mesh(),
    compiler_params=pltpu.CompilerParams(needs_layout_passes=False),
    scratch_types=[
        pltpu.VMEM((_EPT_A,), jnp.int32),
        pltpu.VMEM((N_PAD,), jnp.int32),
        pltpu.VMEM((NS * (_HALF // NS),), jnp.int32),
        pltpu.VMEM_SHARED((NS * N_PAD,), jnp.int32),
        pltpu.SemaphoreType.DMA,
    ],
)
def _deg_kernel(dst_hbm, cnt_hbm, dst_v, acc_v, stg_v, shared, sem_r):
    cid = lax.axis_index("c")
    sid = lax.axis_index("s")
    zeros = jnp.zeros((L,), jnp.int32)

    def zero_body(i, _):
        acc_v[pl.ds(i * L, L)] = zeros

    lax.fori_loop(0, N_PAD // L, zero_body, None, unroll=8)

    pltpu.sync_copy(dst_hbm.at[pl.ds(sid * _EPT_A, _EPT_A)], dst_v)
    ones = jnp.ones((L,), jnp.int32)

    def body(i, _):
        idx = dst_v[pl.ds(i * L, L)]
        plsc.addupdate_scatter(acc_v, [idx], ones)

    lax.fori_loop(0, _EPT_A // L, body, None, unroll=4)

    pltpu.sync_copy(acc_v, shared.at[pl.ds(sid * N_PAD, N_PAD)])
    plsc.subcore_barrier()

    chunk = _HALF // NS
    base = cid * _HALF + sid * chunk

    for g in range(0, NS, 8):
        hs = [pltpu.async_copy(shared.at[pl.ds(t * N_PAD + base, chunk)],
                               stg_v.at[pl.ds(t * chunk, chunk)], sem_r)
              for t in range(g, g + 8)]
        for h in hs:
            h.wait()

    def red_body(i, _):
        v = stg_v[pl.ds(i * L, L)]
        for t in range(1, NS):
            v = v + stg_v[pl.ds(t * chunk + i * L, L)]
        acc_v[pl.ds(i * L, L)] = v

    lax.fori_loop(0, chunk // L, red_body, None, unroll=4)
    pltpu.sync_copy(acc_v.at[pl.ds(0, chunk)], cnt_hbm.at[pl.ds(base, chunk)])


_EPC = N_EDGES // NC
_EPT_B = _EPC // NS
_BB = 80
_NB = _EPT_B // _BB
_ROWS_PER_TILE = N_PAD // NS


@functools.partial(
    pl.kernel,
    out_type=jax.ShapeDtypeStruct((NC, N_PAD, FEAT), jnp.float32),
    mesh=_mesh(),
    compiler_params=pltpu.CompilerParams(needs_layout_passes=False),
    scratch_types=[
        pltpu.VMEM((_EPT_B,), jnp.int32),
        pltpu.VMEM((_BB,), jnp.int32),
        pltpu.VMEM((_BB,), jnp.int32),
        pltpu.VMEM((_BB, FEAT), jnp.float32),
        pltpu.VMEM((_BB, FEAT), jnp.float32),
        pltpu.VMEM_SHARED((N_PAD, FEAT), jnp.float32),
        pltpu.SemaphoreType.DMA,
        pltpu.SemaphoreType.DMA,
        pltpu.SemaphoreType.DMA,
        pltpu.SemaphoreType.DMA,
    ],
)
def _msg_kernel(src_hbm, dst_hbm, p_hbm, out_hbm, src_v, dstb0, dstb1,
                rows0, rows1, acc, sem_i, sem_d, sem_g0, sem_g1):
    cid = lax.axis_index("c")
    sid = lax.axis_index("s")
    ebase = cid * _EPC + sid * _EPT_B

    h_src = pltpu.async_copy(src_hbm.at[pl.ds(ebase, _EPT_B)], src_v, sem_i)

    zeros = jnp.zeros((L,), jnp.float32)

    def zero_body(i, _):
        rows0[i // (FEAT // L), pl.ds((i % (FEAT // L)) * L, L)] = zeros

    lax.fori_loop(0, _BB * FEAT // L, zero_body, None, unroll=8)

    row0 = sid * _ROWS_PER_TILE
    for j in range(_ROWS_PER_TILE // _BB):
        pltpu.sync_copy(rows0, acc.at[pl.ds(row0 + j * _BB, _BB)])
    h_src.wait()
    plsc.subcore_barrier()

    def gather(b, rows, sem):
        return pltpu.async_copy(
            p_hbm.at[src_v.at[pl.ds(b * _BB, _BB)]], rows, sem)

    def gather_wait(b, rows, sem):
        pltpu.make_async_copy(
            p_hbm.at[src_v.at[pl.ds(b * _BB, _BB)]], rows, sem).wait()

    pltpu.async_copy(dst_hbm.at[pl.ds(ebase, _BB)], dstb0, sem_d)
    gather(0, rows0, sem_g0)

    def half(b, dst_cur, dst_nxt, rows_cur, sem_cur, rows_nxt, sem_nxt):
        pltpu.async_copy(dst_hbm.at[pl.ds(ebase + (b + 1) * _BB, _BB)],
                         dst_nxt, sem_d)
        gather(b + 1, rows_nxt, sem_nxt)
        gather_wait(b, rows_cur, sem_cur)
        pltpu.make_async_copy(dst_hbm.at[pl.ds(ebase + b * _BB, _BB)],
                              dst_cur, sem_d).wait()
        pltpu.sync_copy(rows_cur, acc.at[dst_cur], add=True)

    def body(jo, _):
        half(jo * 2, dstb0, dstb1, rows0, sem_g0, rows1, sem_g1)
        half(jo * 2 + 1, dstb1, dstb0, rows1, sem_g1, rows0, sem_g0)

    lax.fori_loop(0, (_NB - 1) // 2, body, None)

    b_last = _NB - 1
    gather_wait(b_last, rows0, sem_g0)
    pltpu.make_async_copy(dst_hbm.at[pl.ds(ebase + b_last * _BB, _BB)],
                          dstb0, sem_d).wait()
    pltpu.sync_copy(rows0, acc.at[dstb0], add=True)

    plsc.subcore_barrier()
    pltpu.sync_copy(acc.at[pl.ds(row0, _ROWS_PER_TILE)],
                    out_hbm.at[cid, pl.ds(row0, _ROWS_PER_TILE)])


_EPT_C = N_EDGES // NS


@functools.partial(
    pl.kernel,
    out_type=jax.ShapeDtypeStruct((N_PAD,), jnp.float32),
    mesh=_mesh(),
    compiler_params=pltpu.CompilerParams(needs_layout_passes=False),
    scratch_types=[
        pltpu.VMEM((_EPT_C,), jnp.int32),
        pltpu.VMEM((N_PAD,), jnp.float32),
        pltpu.VMEM((N_PAD,), jnp.float32),
        pltpu.VMEM((NS * (_HALF // NS),), jnp.float32),
        pltpu.VMEM((_HALF // NS,), jnp.float32),
        pltpu.VMEM((_HALF // NS,), jnp.float32),
        pltpu.VMEM_SHARED((NS * N_PAD,), jnp.float32),
        pltpu.SemaphoreType.DMA,
    ],
)
def _scalar_kernel(pk_hbm, q_hbm, dinv_hbm, r_hbm, out_hbm,
                   pk_v, q_v, acc_v, stg_v, red_v, aux_v, shared, sem_i):
    cid = lax.axis_index("c")
    sid = lax.axis_index("s")
    zeros = jnp.zeros((L,), jnp.float32)

    h_pk = pltpu.async_copy(pk_hbm.at[pl.ds(sid * _EPT_C, _EPT_C)], pk_v, sem_i)
    h_q = pltpu.async_copy(q_hbm, q_v, sem_i)

    def zero_body(i, _):
        acc_v[pl.ds(i * L, L)] = zeros

    lax.fori_loop(0, N_PAD // L, zero_body, None, unroll=8)
    h_pk.wait()
    h_q.wait()

    def body(i, _):
        v = pk_v[pl.ds(i * L, L)]
        s_idx = v & 0xFFFF
        d_idx = v >> 16
        vals = plsc.load_gather(q_v, [s_idx])
        plsc.addupdate_scatter(acc_v, [d_idx], vals)

    lax.fori_loop(0, _EPT_C // L, body, None, unroll=4)

    pltpu.sync_copy(acc_v, shared.at[pl.ds(sid * N_PAD, N_PAD)])
    plsc.subcore_barrier()

    chunk = _HALF // NS
    base = cid * _HALF + sid * chunk

    h_di = pltpu.async_copy(dinv_hbm.at[pl.ds(base, chunk)], red_v, sem_i)
    h_r = pltpu.async_copy(r_hbm.at[pl.ds(base, chunk)], aux_v, sem_i)
    h_di.wait()
    h_r.wait()
    for g in range(0, NS, 8):
        hs = [pltpu.async_copy(shared.at[pl.ds(t * N_PAD + base, chunk)],
                               stg_v.at[pl.ds(t * chunk, chunk)], sem_i)
              for t in range(g, g + 8)]
        for h in hs:
            h.wait()

    def red_body(i, _):
        v = stg_v[pl.ds(i * L, L)]
        for t in range(1, NS):
            v = v + stg_v[pl.ds(t * chunk + i * L, L)]
        acc_v[pl.ds(i * L, L)] = v * red_v[pl.ds(i * L, L)] + aux_v[pl.ds(i * L, L)]

    lax.fori_loop(0, chunk // L, red_body, None, unroll=4)
    pltpu.sync_copy(acc_v.at[pl.ds(0, chunk)], out_hbm.at[pl.ds(base, chunk)])


_RB = 2048
_NRB = N_PAD // _RB


def _tc1_body(cnt_ref, x_ref, w1_ref, dinv_ref, p_ref):
    deg = cnt_ref[...].astype(jnp.float32) + 1.0
    dinv = lax.rsqrt(deg)
    dinv_ref[...] = dinv
    h = jnp.dot(x_ref[...], w1_ref[...], preferred_element_type=jnp.float32)
    p_ref[...] = dinv * h


def _tc1(cnt, x_pad, W1):
    return pl.pallas_call(
        _tc1_body,
        grid=(_NRB,),
        in_specs=[
            pl.BlockSpec((_RB, 1), lambda i: (i, 0)),
            pl.BlockSpec((_RB, FEAT), lambda i: (i, 0)),
            pl.BlockSpec((FEAT, FEAT), lambda i: (0, 0)),
        ],
        out_specs=[
            pl.BlockSpec((_RB, 1), lambda i: (i, 0)),
            pl.BlockSpec((_RB, FEAT), lambda i: (i, 0)),
        ],
        out_shape=[
            jax.ShapeDtypeStruct((N_PAD, 1), jnp.float32),
            jax.ShapeDtypeStruct((N_PAD, FEAT), jnp.float32),
        ],
    )(cnt, x_pad, W1)


def _tc2_body(s_ref, p_ref, dinv_ref, b1_ref, w2_ref, b2_ref, q_ref, r_ref):
    dinv = dinv_ref[...]
    s = s_ref[0] + s_ref[1] + p_ref[...]
    t = jnp.maximum(dinv * s + b1_ref[...], 0.0)
    z = jnp.dot(t, w2_ref[...], preferred_element_type=jnp.float32)
    q = dinv * z
    q_ref[...] = q
    r_ref[...] = dinv * q + b2_ref[0, 0]


def _tc2(s_parts, p, dinv, b1, W2, b2):
    return pl.pallas_call(
        _tc2_body,
        grid=(_NRB,),
        in_specs=[
            pl.BlockSpec((NC, _RB, FEAT), lambda i: (0, i, 0)),
            pl.BlockSpec((_RB, FEAT), lambda i: (i, 0)),
            pl.BlockSpec((_RB, 1), lambda i: (i, 0)),
            pl.BlockSpec((1, FEAT), lambda i: (0, 0)),
            pl.BlockSpec((FEAT, 1), lambda i: (0, 0)),
            pl.BlockSpec((1, 1), lambda i: (0, 0)),
        ],
        out_specs=[
            pl.BlockSpec((_RB, 1), lambda i: (i, 0)),
            pl.BlockSpec((_RB, 1), lambda i: (i, 0)),
        ],
        out_shape=[
            jax.ShapeDtypeStruct((N_PAD, 1), jnp.float32),
            jax.ShapeDtypeStruct((N_PAD, 1), jnp.float32),
        ],
    )(s_parts, p, dinv, b1, W2, b2)


def kernel(x, edge_index, W1, b1, W2, b2):
    edge_index = edge_index.astype(jnp.int32)
    src_idx = edge_index[0]
    dst_idx = edge_index[1]
    x_pad = jnp.pad(x, ((0, N_PAD - N_NODES), (0, 0)))

    pk = jnp.bitwise_or(src_idx, jnp.left_shift(dst_idx, 16))

    cnt = _deg_kernel(dst_idx)
    dinv, p = _tc1(cnt.reshape(N_PAD, 1), x_pad, W1)
    s_parts = _msg_kernel(src_idx, dst_idx, p)
    q, r = _tc2(s_parts, p, dinv, b1.reshape(1, FEAT), W2, b2.reshape(1, 1))
    out = _scalar_kernel(pk, q.reshape(-1), dinv.reshape(-1), r.reshape(-1))
    return out[:N_NODES]


# speedup vs baseline: 3.8978x; 1.0018x over previous
import functools

import jax
import jax.numpy as jnp
from jax import lax
from jax.experimental import pallas as pl
from jax.experimental.pallas import tpu as pltpu
from jax.experimental.pallas import tpu_sc as plsc

N_NODES = 10000
N_PAD = 10240
N_EDGES = 320000
FEAT = 128

NC, NS, L = 2, 16, 16
NW = NC * NS

_mesh = functools.partial(
    plsc.VectorSubcoreMesh, core_axis_name="c", subcore_axis_name="s")


_EPT_A = N_EDGES // NS
_HALF = N_PAD // NC


@functools.partial(
    pl.kernel,
    out_type=jax.ShapeDtypeStruct((N_PAD,), jnp.int32),
    mesh=_mesh(),
    compiler_params=pltpu.CompilerParams(needs_layout_passes=False),
    scratch_types=[
        pltpu.VMEM((_EPT_A,), jnp.int32),
        pltpu.VMEM((N_PAD,), jnp.int32),
        pltpu.VMEM((NS * (_HALF // NS),), jnp.int32),
        pltpu.VMEM_SHARED((NS * N_PAD,), jnp.int32),
        pltpu.SemaphoreType.DMA,
    ],
)
def _deg_kernel(dst_hbm, cnt_hbm, dst_v, acc_v, stg_v, shared, sem_r):
    cid = lax.axis_index("c")
    sid = lax.axis_index("s")
    zeros = jnp.zeros((L,), jnp.int32)

    def zero_body(i, _):
        acc_v[pl.ds(i * L, L)] = zeros

    lax.fori_loop(0, N_PAD // L, zero_body, None, unroll=8)

    pltpu.sync_copy(dst_hbm.at[pl.ds(sid * _EPT_A, _EPT_A)], dst_v)
    ones = jnp.ones((L,), jnp.int32)

    def body(i, _):
        idx = dst_v[pl.ds(i * L, L)]
        plsc.addupdate_scatter(acc_v, [idx], ones)

    lax.fori_loop(0, _EPT_A // L, body, None, unroll=4)

    pltpu.sync_copy(acc_v, shared.at[pl.ds(sid * N_PAD, N_PAD)])
    plsc.subcore_barrier()

    chunk = _HALF // NS
    base = cid * _HALF + sid * chunk

    for g in range(0, NS, 8):
        hs = [pltpu.async_copy(shared.at[pl.ds(t * N_PAD + base, chunk)],
                               stg_v.at[pl.ds(t * chunk, chunk)], sem_r)
              for t in range(g, g + 8)]
        for h in hs:
            h.wait()

    def red_body(i, _):
        v = stg_v[pl.ds(i * L, L)]
        for t in range(1, NS):
            v = v + stg_v[pl.ds(t * chunk + i * L, L)]
        acc_v[pl.ds(i * L, L)] = v

    lax.fori_loop(0, chunk // L, red_body, None, unroll=4)
    pltpu.sync_copy(acc_v.at[pl.ds(0, chunk)], cnt_hbm.at[pl.ds(base, chunk)])


_EPC = N_EDGES // NC
_EPT_B = _EPC // NS
_BB = 80
_NB = _EPT_B // _BB
_ROWS_PER_TILE = N_PAD // NS


@functools.partial(
    pl.kernel,
    out_type=jax.ShapeDtypeStruct((NC, N_PAD, FEAT), jnp.float32),
    mesh=_mesh(),
    compiler_params=pltpu.CompilerParams(needs_layout_passes=False),
    scratch_types=[
        pltpu.VMEM((_EPT_B,), jnp.int32),
        pltpu.VMEM((_BB,), jnp.int32),
        pltpu.VMEM((_BB,), jnp.int32),
        pltpu.VMEM((_BB, FEAT), jnp.float32),
        pltpu.VMEM((_BB, FEAT), jnp.float32),
        pltpu.VMEM_SHARED((N_PAD, FEAT), jnp.float32),
        pltpu.SemaphoreType.DMA,
        pltpu.SemaphoreType.DMA,
        pltpu.SemaphoreType.DMA,
        pltpu.SemaphoreType.DMA,
        pltpu.SemaphoreType.DMA,
        pltpu.SemaphoreType.DMA,
    ],
)
def _msg_kernel(src_hbm, dst_hbm, p_hbm, out_hbm, src_v, dstb0, dstb1,
                rows0, rows1, acc, sem_i, sem_d, sem_g0, sem_g1,
                sem_s0, sem_s1):
    cid = lax.axis_index("c")
    sid = lax.axis_index("s")
    ebase = cid * _EPC + sid * _EPT_B

    h_src = pltpu.async_copy(src_hbm.at[pl.ds(ebase, _EPT_B)], src_v, sem_i)

    zeros = jnp.zeros((L,), jnp.float32)

    def zero_body(i, _):
        rows0[i // (FEAT // L), pl.ds((i % (FEAT // L)) * L, L)] = zeros

    lax.fori_loop(0, _BB * FEAT // L, zero_body, None, unroll=8)

    row0 = sid * _ROWS_PER_TILE
    for j in range(_ROWS_PER_TILE // _BB):
        pltpu.sync_copy(rows0, acc.at[pl.ds(row0 + j * _BB, _BB)])
    h_src.wait()
    plsc.subcore_barrier()

    def gather(b, rows, sem):
        return pltpu.async_copy(
            p_hbm.at[src_v.at[pl.ds(b * _BB, _BB)]], rows, sem)

    def gather_wait(b, rows, sem):
        pltpu.make_async_copy(
            p_hbm.at[src_v.at[pl.ds(b * _BB, _BB)]], rows, sem).wait()

    def scatter(rows, dst, sem):
        return pltpu.async_copy(rows, acc.at[dst], sem, add=True)

    def scatter_wait(rows, dst, sem):
        pltpu.make_async_copy(rows, acc.at[dst], sem).wait()

    def dst_wait(b, dst):
        pltpu.make_async_copy(dst_hbm.at[pl.ds(ebase + b * _BB, _BB)],
                              dst, sem_d).wait()

    pltpu.async_copy(dst_hbm.at[pl.ds(ebase, _BB)], dstb0, sem_d)
    gather(0, rows0, sem_g0)

    def half(b, dst_cur, dst_nxt, rows_cur, sem_cur, sem_scur,
             rows_nxt, sem_nxt, sem_snxt, first):
        if not first:
            scatter_wait(rows_nxt, dst_nxt, sem_snxt)
        pltpu.async_copy(dst_hbm.at[pl.ds(ebase + (b + 1) * _BB, _BB)],
                         dst_nxt, sem_d)
        gather(b + 1, rows_nxt, sem_nxt)
        gather_wait(b, rows_cur, sem_cur)
        dst_wait(b, dst_cur)
        scatter(rows_cur, dst_cur, sem_scur)

    half(0, dstb0, dstb1, rows0, sem_g0, sem_s0, rows1, sem_g1, sem_s1, True)
    half(1, dstb1, dstb0, rows1, sem_g1, sem_s1, rows0, sem_g0, sem_s0, False)

    def body(jo, _):
        half(jo * 2, dstb0, dstb1, rows0, sem_g0, sem_s0,
             rows1, sem_g1, sem_s1, False)
        half(jo * 2 + 1, dstb1, dstb0, rows1, sem_g1, sem_s1,
             rows0, sem_g0, sem_s0, False)

    lax.fori_loop(1, (_NB - 1) // 2, body, None)

    b_last = _NB - 1
    scatter_wait(rows1, dstb1, sem_s1)
    gather_wait(b_last, rows0, sem_g0)
    dst_wait(b_last, dstb0)
    pltpu.sync_copy(rows0, acc.at[dstb0], add=True)

    plsc.subcore_barrier()
    pltpu.sync_copy(acc.at[pl.ds(row0, _ROWS_PER_TILE)],
                    out_hbm.at[cid, pl.ds(row0, _ROWS_PER_TILE)])


_EPT_C = N_EDGES // NS


@functools.partial(
    pl.kernel,
    out_type=jax.ShapeDtypeStruct((N_PAD,), jnp.float32),
    mesh=_mesh(),
    compiler_params=pltpu.CompilerParams(needs_layout_passes=False),
    scratch_types=[
        pltpu.VMEM((_EPT_C,), jnp.int32),
        pltpu.VMEM((N_PAD,), jnp.float32),
        pltpu.VMEM((N_PAD,), jnp.float32),
        pltpu.VMEM((NS * (_HALF // NS),), jnp.float32),
        pltpu.VMEM((_HALF // NS,), jnp.float32),
        pltpu.VMEM((_HALF // NS,), jnp.float32),
        pltpu.VMEM_SHARED((NS * N_PAD,), jnp.float32),
        pltpu.SemaphoreType.DMA,
    ],
)
def _scalar_kernel(pk_hbm, q_hbm, dinv_hbm, r_hbm, out_hbm,
                   pk_v, q_v, acc_v, stg_v, red_v, aux_v, shared, sem_i):
    cid = lax.axis_index("c")
    sid = lax.axis_index("s")
    zeros = jnp.zeros((L,), jnp.float32)

    h_pk = pltpu.async_copy(pk_hbm.at[pl.ds(sid * _EPT_C, _EPT_C)], pk_v, sem_i)
    h_q = pltpu.async_copy(q_hbm, q_v, sem_i)

    def zero_body(i, _):
        acc_v[pl.ds(i * L, L)] = zeros

    lax.fori_loop(0, N_PAD // L, zero_body, None, unroll=8)
    h_pk.wait()
    h_q.wait()

    def body(i, _):
        v = pk_v[pl.ds(i * L, L)]
        s_idx = v & 0xFFFF
        d_idx = v >> 16
        vals = plsc.load_gather(q_v, [s_idx])
        plsc.addupdate_scatter(acc_v, [d_idx], vals)

    lax.fori_loop(0, _EPT_C // L, body, None, unroll=4)

    pltpu.sync_copy(acc_v, shared.at[pl.ds(sid * N_PAD, N_PAD)])
    plsc.subcore_barrier()

    chunk = _HALF // NS
    base = cid * _HALF + sid * chunk

    h_di = pltpu.async_copy(dinv_hbm.at[pl.ds(base, chunk)], red_v, sem_i)
    h_r = pltpu.async_copy(r_hbm.at[pl.ds(base, chunk)], aux_v, sem_i)
    h_di.wait()
    h_r.wait()
    for g in range(0, NS, 8):
        hs = [pltpu.async_copy(shared.at[pl.ds(t * N_PAD + base, chunk)],
                               stg_v.at[pl.ds(t * chunk, chunk)], sem_i)
              for t in range(g, g + 8)]
        for h in hs:
            h.wait()

    def red_body(i, _):
        v = stg_v[pl.ds(i * L, L)]
        for t in range(1, NS):
            v = v + stg_v[pl.ds(t * chunk + i * L, L)]
        acc_v[pl.ds(i * L, L)] = v * red_v[pl.ds(i * L, L)] + aux_v[pl.ds(i * L, L)]

    lax.fori_loop(0, chunk // L, red_body, None, unroll=4)
    pltpu.sync_copy(acc_v.at[pl.ds(0, chunk)], out_hbm.at[pl.ds(base, chunk)])


_RB = 2048
_NRB = N_PAD // _RB


def _tc1_body(cnt_ref, x_ref, w1_ref, dinv_ref, p_ref):
    deg = cnt_ref[...].astype(jnp.float32) + 1.0
    dinv = lax.rsqrt(deg)
    dinv_ref[...] = dinv
    h = jnp.dot(x_ref[...], w1_ref[...], preferred_element_type=jnp.float32)
    p_ref[...] = dinv * h


def _tc1(cnt, x_pad, W1):
    return pl.pallas_call(
        _tc1_body,
        grid=(_NRB,),
        in_specs=[
            pl.BlockSpec((_RB, 1), lambda i: (i, 0)),
            pl.BlockSpec((_RB, FEAT), lambda i: (i, 0)),
            pl.BlockSpec((FEAT, FEAT), lambda i: (0, 0)),
        ],
        out_specs=[
            pl.BlockSpec((_RB, 1), lambda i: (i, 0)),
            pl.BlockSpec((_RB, FEAT), lambda i: (i, 0)),
        ],
        out_shape=[
            jax.ShapeDtypeStruct((N_PAD, 1), jnp.float32),
            jax.ShapeDtypeStruct((N_PAD, FEAT), jnp.float32),
        ],
    )(cnt, x_pad, W1)


def _tc2_body(s_ref, p_ref, dinv_ref, b1_ref, w2_ref, b2_ref, q_ref, r_ref):
    dinv = dinv_ref[...]
    s = s_ref[0] + s_ref[1] + p_ref[...]
    t = jnp.maximum(dinv * s + b1_ref[...], 0.0)
    z = jnp.dot(t, w2_ref[...], preferred_element_type=jnp.float32)
    q = dinv * z
    q_ref[...] = q
    r_ref[...] = dinv * q + b2_ref[0, 0]


def _tc2(s_parts, p, dinv, b1, W2, b2):
    return pl.pallas_call(
        _tc2_body,
        grid=(_NRB,),
        in_specs=[
            pl.BlockSpec((NC, _RB, FEAT), lambda i: (0, i, 0)),
            pl.BlockSpec((_RB, FEAT), lambda i: (i, 0)),
            pl.BlockSpec((_RB, 1), lambda i: (i, 0)),
            pl.BlockSpec((1, FEAT), lambda i: (0, 0)),
            pl.BlockSpec((FEAT, 1), lambda i: (0, 0)),
            pl.BlockSpec((1, 1), lambda i: (0, 0)),
        ],
        out_specs=[
            pl.BlockSpec((_RB, 1), lambda i: (i, 0)),
            pl.BlockSpec((_RB, 1), lambda i: (i, 0)),
        ],
        out_shape=[
            jax.ShapeDtypeStruct((N_PAD, 1), jnp.float32),
            jax.ShapeDtypeStruct((N_PAD, 1), jnp.float32),
        ],
    )(s_parts, p, dinv, b1, W2, b2)


def kernel(x, edge_index, W1, b1, W2, b2):
    edge_index = edge_index.astype(jnp.int32)
    src_idx = edge_index[0]
    dst_idx = edge_index[1]
    x_pad = jnp.pad(x, ((0, N_PAD - N_NODES), (0, 0)))

    pk = jnp.bitwise_or(src_idx, jnp.left_shift(dst_idx, 16))

    cnt = _deg_kernel(dst_idx)
    dinv, p = _tc1(cnt.reshape(N_PAD, 1), x_pad, W1)
    s_parts = _msg_kernel(src_idx, dst_idx, p)
    q, r = _tc2(s_parts, p, dinv, b1.reshape(1, FEAT), W2, b2.reshape(1, 1))
    out = _scalar_kernel(pk, q.reshape(-1), dinv.reshape(-1), r.reshape(-1))
    return out[:N_NODES]
